# baseline (speedup 1.0000x reference)
import functools

import jax
import jax.numpy as jnp
from jax import lax
from jax.experimental import pallas as pl
from jax.experimental.pallas import tpu as pltpu
from jax.experimental.pallas import tpu_sc as plsc

VOCAB = 1000000
D = 64
B = 16384
NC = 2
NS = 16
NW = NC * NS
BPW = B // NW
CH = 128
NCH = BPW // CH
LANES = 16
PR = 2 * D


@functools.cache
def _build_encoder():
    mesh = plsc.VectorSubcoreMesh(core_axis_name="c", subcore_axis_name="s")

    @functools.partial(
        pl.kernel,
        mesh=mesh,
        out_type=jax.ShapeDtypeStruct((B // 2, PR), jnp.float32),
        scratch_types=[
            pltpu.VMEM((NCH, CH), jnp.int32),
            pltpu.VMEM((NCH, CH), jnp.int32),
            pltpu.VMEM((2, CH, PR), jnp.float32),
            pltpu.VMEM((2, CH, PR), jnp.float32),
            pltpu.VMEM((BPW // 2, PR), jnp.float32),
            pltpu.SemaphoreType.DMA,
            pltpu.SemaphoreType.DMA,
        ],
        compiler_params=pltpu.CompilerParams(needs_layout_passes=False),
    )
    def _encode(idxp_hbm, hb_hbm, item_hbm, ent_hbm, out_hbm,
                idxp_v, hb_v, a_v, b_v, out_v, sem_a, sem_b):
        wid = lax.axis_index("s") * NC + lax.axis_index("c")

        pltpu.sync_copy(idxp_hbm.at[pl.ds(wid * NCH, NCH)], idxp_v)
        pltpu.sync_copy(hb_hbm.at[pl.ds(wid * NCH, NCH)], hb_v)

        def fire(j, buf):
            ca = pltpu.async_copy(item_hbm.at[idxp_v.at[j]], a_v.at[buf], sem_a)
            cb = pltpu.async_copy(ent_hbm.at[idxp_v.at[j]], b_v.at[buf], sem_b)
            return ca, cb

        lane = lax.iota(jnp.int32, LANES)

        def repack(j, buf):
            aj = a_v.at[buf]
            bj = b_v.at[buf]

            def row_body(r, carry):
                hb16 = plsc.load_gather(hb_v, [jnp.full((LANES,), j, jnp.int32),
                                               jnp.full((LANES,), r, jnp.int32)])
                rr = jnp.full((LANES,), r, jnp.int32)
                g = j * CH + r
                q = g // 2
                co = (g % 2) * D
                for cg in range(D // LANES):
                    col = hb16 + (cg * LANES) + lane
                    va = plsc.load_gather(aj, [rr, col])
                    vb = plsc.load_gather(bj, [rr, col])
                    out_v[q, pl.ds(co + cg * LANES, LANES)] = va + vb
                return carry

            lax.fori_loop(0, CH, row_body, 0)

        cops = fire(0, 0)
        for j in range(NCH):
            nxt = fire(j + 1, (j + 1) % 2) if j + 1 < NCH else None
            for c in cops:
                c.wait()
            repack(j, j % 2)
            cops = nxt

        pltpu.sync_copy(out_v, out_hbm.at[pl.ds(wid * (BPW // 2), BPW // 2)])

    return _encode


def kernel(batch_data, item_table, ent_table):
    idxp = (batch_data >> 1).reshape(NW * NCH, CH)
    hb = ((batch_data & 1) * D).reshape(NW * NCH, CH)
    item2 = item_table.reshape(VOCAB // 2, PR)
    ent2 = ent_table.reshape(VOCAB // 2, PR)
    out2 = _build_encoder()(idxp, hb, item2, ent2)
    return out2.reshape(B, D)
